# baseline (speedup 1.0000x reference)
import functools

import jax
import jax.numpy as jnp
from jax import lax
from jax.experimental import pallas as pl
from jax.experimental.pallas import tpu as pltpu
from jax.experimental.pallas import tpu_sc as plsc

B, S = 1024, 26
NUM_CLASSES = 1000
NC, NS, L = 2, 16, 16
NW = NC * NS
B_PER_W = B // NW
CHUNK_B = 2
CHUNK_R = CHUNK_B * S
N_CHUNKS = B_PER_W // CHUNK_B
IDX_PAD = 64
IDX_PER_W = N_CHUNKS * IDX_PAD


def _body(x_hbm, out_hbm, idx_v, buf_v, sem):
    wid = lax.axis_index("c") * NS + lax.axis_index("s")
    batch0 = wid * B_PER_W

    pltpu.sync_copy(x_hbm.at[pl.ds(wid * IDX_PER_W, IDX_PER_W)], idx_v)

    zeros = jnp.zeros((L,), jnp.float32)
    ones = jnp.ones((L,), jnp.float32)
    lane = lax.iota(jnp.int32, L)

    def zero_row(r, _):
        b = r // S
        rr = r % S
        def zero_slice(i, _):
            buf_v[b, rr, pl.ds(i * L, L)] = zeros
            return _
        lax.fori_loop(0, NUM_CLASSES // L, zero_slice, 0)
        buf_v[b, rr, pl.ds(NUM_CLASSES - L, L)] = zeros
        return _
    lax.fori_loop(0, CHUNK_R, zero_row, 0)

    def chunk_body(ci, _):
        def scatter(vals):
            for j in range(CHUNK_R // L + 1):
                r = lane + j * L
                mask = r < CHUNK_R
                cols = idx_v[pl.ds(ci * IDX_PAD + j * L, L)]
                plsc.store_scatter(buf_v, [r // S, r % S, cols], vals, mask=mask)
        scatter(ones)
        pltpu.sync_copy(buf_v, out_hbm.at[pl.ds(batch0 + ci * CHUNK_B, CHUNK_B)])
        scatter(zeros)
        return _
    lax.fori_loop(0, N_CHUNKS, chunk_body, 0)


@functools.partial(jax.jit, static_argnames=())
def kernel(x, eye):
    del eye
    xr = x.reshape(NW, N_CHUNKS, CHUNK_R)
    xp = jnp.pad(xr, ((0, 0), (0, 0), (0, IDX_PAD - CHUNK_R))).reshape(-1)
    mesh = plsc.VectorSubcoreMesh(core_axis_name="c", subcore_axis_name="s")
    k = pl.kernel(
        _body,
        out_type=jax.ShapeDtypeStruct((B, S, NUM_CLASSES), jnp.float32),
        mesh=mesh,
        scratch_types=[
            pltpu.VMEM((IDX_PER_W,), jnp.int32),
            pltpu.VMEM((CHUNK_B, S, NUM_CLASSES), jnp.float32),
            pltpu.SemaphoreType.DMA,
        ],
        compiler_params=pltpu.CompilerParams(
            needs_layout_passes=False, use_tc_tiling_on_sc=True),
    )
    return k(xp)


# speedup vs baseline: 2.2108x; 2.2108x over previous
import functools

import jax
import jax.numpy as jnp
from jax import lax
from jax.experimental import pallas as pl
from jax.experimental.pallas import tpu as pltpu
from jax.experimental.pallas import tpu_sc as plsc

B, S = 1024, 26
NUM_CLASSES = 1000
NC, NS, L = 2, 16, 16
NW = NC * NS
CC = 40
CPS = NUM_CLASSES // CC
UNITS = S * CPS
BV = B // L


def _body(x_hbm, out_hbm, idx_v, buf_v):
    wid = lax.axis_index("c") * NS + lax.axis_index("s")
    u0 = wid * UNITS // NW
    u1 = (wid + 1) * UNITS // NW

    s_base = u0 // CPS
    pltpu.sync_copy(x_hbm.at[pl.ds(s_base * B, 2 * B)], idx_v)

    zeros = jnp.zeros((L,), jnp.float32)
    ones = jnp.ones((L,), jnp.float32)
    lane = lax.iota(jnp.int32, L)

    def zero_row(r, c):
        def zero_vec(k, c):
            buf_v[r, pl.ds(k * L, L)] = zeros
            return c
        return lax.fori_loop(0, BV, zero_vec, c)
    lax.fori_loop(0, CC, zero_row, 0)

    def unit_body(u, c):
        s_off = u // CPS - s_base
        c0 = (u % CPS) * CC
        def scan(vals):
            for k in range(BV):
                ivec = idx_v[pl.ds(s_off * B + k * L, L)]
                m = (ivec >= c0) & (ivec < c0 + CC)
                plsc.store_scatter(buf_v, [ivec - c0, lane + k * L], vals,
                                   mask=m)
        scan(ones)
        pltpu.sync_copy(buf_v, out_hbm.at[u // CPS, pl.ds(c0, CC)])
        scan(zeros)
        return c
    lax.fori_loop(u0, u1, unit_body, 0)


@functools.partial(jax.jit, static_argnames=())
def kernel(x, eye):
    del eye
    xt = jnp.pad(x.T, ((0, 1), (0, 0))).reshape(-1)
    mesh = plsc.VectorSubcoreMesh(core_axis_name="c", subcore_axis_name="s",
                                  num_cores=NC, num_subcores=NS)
    k = pl.kernel(
        _body,
        out_type=jax.ShapeDtypeStruct((S, NUM_CLASSES, B), jnp.float32),
        mesh=mesh,
        scratch_types=[
            pltpu.VMEM((2 * B,), jnp.int32),
            pltpu.VMEM((CC, B), jnp.float32),
        ],
        compiler_params=pltpu.CompilerParams(needs_layout_passes=False),
    )
    out = k(xt)
    return jnp.transpose(out, (2, 0, 1))


# speedup vs baseline: 2.5646x; 1.1600x over previous
import functools

import jax
import jax.numpy as jnp
from jax import lax
from jax.experimental import pallas as pl
from jax.experimental.pallas import tpu as pltpu
from jax.experimental.pallas import tpu_sc as plsc

B, S = 1024, 26
NUM_CLASSES = 1000
NC, NS, L = 2, 16, 16
NW = NC * NS
CC = 40
CPS = NUM_CLASSES // CC
UNITS = S * CPS
BV = B // L


def _body(x_hbm, out_hbm, idx_v, buf_v, sem0, sem1):
    wid = lax.axis_index("c") * NS + lax.axis_index("s")
    u0 = wid * UNITS // NW
    u1 = (wid + 1) * UNITS // NW
    sems = (sem0, sem1)

    s_base = u0 // CPS
    pltpu.sync_copy(x_hbm.at[pl.ds(s_base * B, 2 * B)], idx_v)

    zeros = jnp.zeros((L,), jnp.float32)
    ones = jnp.ones((L,), jnp.float32)
    lane = lax.iota(jnp.int32, L)

    def zero_row(r, c):
        def zero_vec(k, c):
            buf_v[r // CC, r % CC, pl.ds(k * L, L)] = zeros
            return c
        return lax.fori_loop(0, BV, zero_vec, c)
    lax.fori_loop(0, 2 * CC, zero_row, 0)

    def scan(slot, u, vals):
        s_off = u // CPS - s_base
        c0 = (u % CPS) * CC
        slot_vec = jnp.full((L,), slot, jnp.int32)
        def scan_vec(k, c):
            ivec = idx_v[pl.ds(s_off * B + k * L, L)]
            m = (ivec >= c0) & (ivec < c0 + CC)
            plsc.store_scatter(buf_v, [slot_vec, ivec - c0, lane + k * L],
                               vals, mask=m)
            return c
        lax.fori_loop(0, BV, scan_vec, 0)

    def drain(slot):
        pltpu.make_async_copy(out_hbm.at[0, pl.ds(0, CC)], buf_v.at[slot],
                              sems[slot]).wait()

    n_groups = (u1 - u0 + 1) // 2
    def group(g, c):
        for slot in (0, 1):
            u = u0 + g * 2 + slot
            @pl.when(u < u1)
            def _issue():
                @pl.when(g > 0)
                def _reclaim():
                    drain(slot)
                    scan(slot, u - 2, zeros)
                scan(slot, u, ones)
                pltpu.make_async_copy(
                    buf_v.at[slot],
                    out_hbm.at[u // CPS, pl.ds((u % CPS) * CC, CC)],
                    sems[slot]).start()
        return c
    lax.fori_loop(0, n_groups, group, 0)
    drain(0)
    drain(1)


@functools.partial(jax.jit, static_argnames=())
def kernel(x, eye):
    del eye
    xt = jnp.pad(x.T, ((0, 1), (0, 0))).reshape(-1)
    mesh = plsc.VectorSubcoreMesh(core_axis_name="c", subcore_axis_name="s",
                                  num_cores=NC, num_subcores=NS)
    k = pl.kernel(
        _body,
        out_type=jax.ShapeDtypeStruct((S, NUM_CLASSES, B), jnp.float32),
        mesh=mesh,
        scratch_types=[
            pltpu.VMEM((2 * B,), jnp.int32),
            pltpu.VMEM((2, CC, B), jnp.float32),
            pltpu.SemaphoreType.DMA,
            pltpu.SemaphoreType.DMA,
        ],
        compiler_params=pltpu.CompilerParams(needs_layout_passes=False),
    )
    out = k(xt)
    return jnp.transpose(out, (2, 0, 1))
